# baseline (speedup 1.0000x reference)
import functools

import jax
import jax.numpy as jnp
from jax.experimental import pallas as pl
from jax.experimental.pallas import tpu as pltpu

LOAD_COEFF = 0.01
TMG = 256


def _router_kernel(flat_ref, wr_ref, u_ref, spec_ref, gen_ref,
                   eid_ref, p2_ref, bal_ref):
    flat = flat_ref[...]
    logits = jax.lax.dot_general(
        flat, wr_ref[...], (((1,), (1,)), ((), ())),
        preferred_element_type=jnp.float32,
    )
    u = jnp.clip(u_ref[...], 0.0, 1.0)
    logits = logits + u * spec_ref[...] + (1.0 - u) * gen_ref[...]
    m = jnp.max(logits, axis=-1, keepdims=True)
    ex = jnp.exp(logits - m)
    sm = ex / jnp.sum(ex, axis=-1, keepdims=True)

    bt, e = sm.shape
    iota = jax.lax.broadcasted_iota(jnp.int32, (bt, e), 1)
    p0 = jnp.max(sm, axis=-1, keepdims=True)
    i0 = jnp.min(jnp.where(sm == p0, iota, e), axis=-1, keepdims=True)
    mask0 = iota == i0
    sm2 = jnp.where(mask0, -1.0, sm)
    p1 = jnp.max(sm2, axis=-1, keepdims=True)
    i1 = jnp.min(jnp.where(sm2 == p1, iota, e), axis=-1, keepdims=True)

    eid_ref[...] = jnp.concatenate([i0, i1], axis=1)
    p2_ref[...] = jnp.concatenate([p0, p1], axis=1)

    mean_sm = jnp.mean(sm, axis=0, keepdims=True)
    bal_ref[...] = jnp.sum(mean_sm * mean_sm, axis=1,
                           keepdims=True) * (e * LOAD_COEFF)


def _ffn_kernel(info_ref, xg_ref, wg_ref, wu_ref, wd_ref, y_ref,
                *, n_expert_tiles, n_tiles):
    m = pl.program_id(0)
    i = pl.program_id(1)
    n_act = info_ref[n_tiles]
    active = jnp.logical_or(i >= n_expert_tiles, i < n_act)

    @pl.when(active)
    def _():
        x = xg_ref[...]
        g = jax.lax.dot_general(x, wg_ref[0], (((1,), (1,)), ((), ())),
                                preferred_element_type=jnp.float32)
        u = jax.lax.dot_general(x, wu_ref[0], (((1,), (1,)), ((), ())),
                                preferred_element_type=jnp.float32)
        h = g * jax.lax.logistic(g) * u
        y = jax.lax.dot_general(h, wd_ref[0], (((1,), (1,)), ((), ())),
                                preferred_element_type=jnp.float32)
        row = i * xg_ref.shape[0]

        @pl.when(m == 0)
        def _():
            y_ref[pl.ds(row, xg_ref.shape[0]), :] = y

        @pl.when(m != 0)
        def _():
            y_ref[pl.ds(row, xg_ref.shape[0]), :] += y


def kernel(x, U, W_r, spec_bias, gen_bias, Wg, Wu, Wd, Sg, Su, Sd):
    B, T, D = x.shape
    E, MID, _ = Wg.shape
    SH = Sg.shape[0]
    BT = B * T
    NS = BT * 2

    flat = x.reshape(BT, D)
    eid2, p2, bal = pl.pallas_call(
        _router_kernel,
        out_shape=[
            jax.ShapeDtypeStruct((BT, 2), jnp.int32),
            jax.ShapeDtypeStruct((BT, 2), jnp.float32),
            jax.ShapeDtypeStruct((1, 1), jnp.float32),
        ],
    )(flat, W_r, U.reshape(BT, 1), spec_bias.reshape(1, E),
      gen_bias.reshape(1, E))

    n_expert_tiles = NS // TMG + (E - 1)
    n_shared_tiles = (BT * SH) // TMG
    n_tiles = n_expert_tiles + n_shared_tiles
    shared_base = n_expert_tiles * TMG
    padmax = n_tiles * TMG

    eid_flat = eid2.reshape(NS)
    onehot = (eid_flat[:, None]
              == jnp.arange(E, dtype=jnp.int32)[None, :]).astype(jnp.int32)
    incl = jnp.cumsum(onehot, axis=0)
    rank = jnp.sum((incl - onehot) * onehot, axis=1)
    counts = incl[-1]
    tiles_e = (counts + TMG - 1) // TMG
    bounds = jnp.cumsum(tiles_e)
    padded_off = (bounds - tiles_e) * TMG
    n_act = bounds[E - 1]

    padpos = jnp.sum(onehot * padded_off[None, :], axis=1) + rank
    slot_tok = jnp.arange(NS, dtype=jnp.int32) // 2
    tok_pad = jnp.zeros((padmax,), jnp.int32).at[padpos].set(slot_tok)
    tok_pad = tok_pad.at[shared_base:].set(
        jnp.tile(jnp.arange(BT, dtype=jnp.int32), SH))
    pos2 = padpos.reshape(BT, 2)

    tile_ids = jnp.arange(n_expert_tiles, dtype=jnp.int32)
    te = jnp.sum((tile_ids[:, None] >= bounds[None, :]).astype(jnp.int32),
                 axis=1)
    te_last = jnp.sum((bounds <= n_act - 1).astype(jnp.int32))
    te = jnp.where(tile_ids < n_act, te, te_last)
    sh_te = E + jnp.arange(n_shared_tiles, dtype=jnp.int32) // (BT // TMG)
    info = jnp.concatenate([te, sh_te, n_act.reshape(1)])

    xg = flat[tok_pad]

    WgA = jnp.concatenate([Wg, Sg], axis=0)
    WuA = jnp.concatenate([Wu, Su], axis=0)
    WdA = jnp.concatenate([Wd, Sd], axis=0)

    MIDC = 512
    n_mid = MID // MIDC
    grid_spec = pltpu.PrefetchScalarGridSpec(
        num_scalar_prefetch=1,
        grid=(n_mid, n_tiles),
        in_specs=[
            pl.BlockSpec((TMG, D), lambda m, i, info: (i, 0)),
            pl.BlockSpec((1, MIDC, D), lambda m, i, info: (info[i], m, 0)),
            pl.BlockSpec((1, MIDC, D), lambda m, i, info: (info[i], m, 0)),
            pl.BlockSpec((1, D, MIDC), lambda m, i, info: (info[i], 0, m)),
        ],
        out_specs=pl.BlockSpec((padmax, D), lambda m, i, info: (0, 0)),
    )
    y = pl.pallas_call(
        functools.partial(_ffn_kernel, n_expert_tiles=n_expert_tiles,
                          n_tiles=n_tiles),
        grid_spec=grid_spec,
        out_shape=jax.ShapeDtypeStruct((padmax, D), jnp.float32),
    )(info, xg, WgA, WuA, WdA)

    out = p2[:, 0:1] * y[pos2[:, 0]] + p2[:, 1:2] * y[pos2[:, 1]]
    for si in range(SH):
        base = shared_base + si * BT
        out = out + y[base:base + BT]

    return out.reshape(B, T, D), bal[0, 0]


# speedup vs baseline: 1.1969x; 1.1969x over previous
import functools

import jax
import jax.numpy as jnp
from jax.experimental import pallas as pl
from jax.experimental.pallas import tpu as pltpu

LOAD_COEFF = 0.01
TMG_MAX = 512


def _router_kernel(flat_ref, wr_ref, u_ref, spec_ref, gen_ref,
                   eid_ref, p2_ref, bal_ref):
    flat = flat_ref[...]
    logits = jax.lax.dot_general(
        flat, wr_ref[...], (((1,), (1,)), ((), ())),
        preferred_element_type=jnp.float32,
    )
    u = jnp.clip(u_ref[...], 0.0, 1.0)
    logits = logits + u * spec_ref[...] + (1.0 - u) * gen_ref[...]
    m = jnp.max(logits, axis=-1, keepdims=True)
    ex = jnp.exp(logits - m)
    sm = ex / jnp.sum(ex, axis=-1, keepdims=True)

    bt, e = sm.shape
    iota = jax.lax.broadcasted_iota(jnp.int32, (bt, e), 1)
    p0 = jnp.max(sm, axis=-1, keepdims=True)
    i0 = jnp.min(jnp.where(sm == p0, iota, e), axis=-1, keepdims=True)
    mask0 = iota == i0
    sm2 = jnp.where(mask0, -1.0, sm)
    p1 = jnp.max(sm2, axis=-1, keepdims=True)
    i1 = jnp.min(jnp.where(sm2 == p1, iota, e), axis=-1, keepdims=True)

    eid_ref[...] = jnp.concatenate([i0, i1], axis=1)
    p2_ref[...] = jnp.concatenate([p0, p1], axis=1)

    mean_sm = jnp.mean(sm, axis=0, keepdims=True)
    bal_ref[...] = jnp.sum(mean_sm * mean_sm, axis=1,
                           keepdims=True) * (e * LOAD_COEFF)


def _ffn_kernel(info_ref, xg_ref, wg_ref, wu_ref, wd_ref, y_ref,
                *, n_expert_tiles, n_tiles):
    i = pl.program_id(0)
    n_act = info_ref[n_tiles]
    active = jnp.logical_or(i >= n_expert_tiles, i < n_act)

    @pl.when(active)
    def _():
        x = xg_ref[...]
        g = jax.lax.dot_general(x, wg_ref[0], (((1,), (1,)), ((), ())),
                                preferred_element_type=jnp.float32)
        u = jax.lax.dot_general(x, wu_ref[0], (((1,), (1,)), ((), ())),
                                preferred_element_type=jnp.float32)
        h = g * jax.lax.logistic(g) * u
        y_ref[...] = jax.lax.dot_general(
            h, wd_ref[0], (((1,), (1,)), ((), ())),
            preferred_element_type=jnp.float32)


def kernel(x, U, W_r, spec_bias, gen_bias, Wg, Wu, Wd, Sg, Su, Sd):
    B, T, D = x.shape
    E, MID, _ = Wg.shape
    SH = Sg.shape[0]
    BT = B * T
    NS = BT * 2
    TMG = min(TMG_MAX, BT)

    flat = x.reshape(BT, D)
    eid2, p2, bal = pl.pallas_call(
        _router_kernel,
        out_shape=[
            jax.ShapeDtypeStruct((BT, 2), jnp.int32),
            jax.ShapeDtypeStruct((BT, 2), jnp.float32),
            jax.ShapeDtypeStruct((1, 1), jnp.float32),
        ],
    )(flat, W_r, U.reshape(BT, 1), spec_bias.reshape(1, E),
      gen_bias.reshape(1, E))

    n_expert_tiles = NS // TMG + (E - 1)
    n_shared_tiles = (BT * SH) // TMG
    n_tiles = n_expert_tiles + n_shared_tiles
    shared_base = n_expert_tiles * TMG
    padmax = n_tiles * TMG

    eid_flat = eid2.reshape(NS)
    onehot = (eid_flat[:, None]
              == jnp.arange(E, dtype=jnp.int32)[None, :]).astype(jnp.int32)
    incl = jnp.cumsum(onehot, axis=0)
    rank = jnp.sum((incl - onehot) * onehot, axis=1)
    counts = incl[-1]
    tiles_e = (counts + TMG - 1) // TMG
    bounds = jnp.cumsum(tiles_e)
    padded_off = (bounds - tiles_e) * TMG
    n_act = bounds[E - 1]

    padpos = jnp.sum(onehot * padded_off[None, :], axis=1) + rank
    slot_tok = jnp.arange(NS, dtype=jnp.int32) // 2
    tok_pad = jnp.zeros((padmax,), jnp.int32).at[padpos].set(slot_tok)
    tok_pad = tok_pad.at[shared_base:].set(
        jnp.tile(jnp.arange(BT, dtype=jnp.int32), SH))
    pos2 = padpos.reshape(BT, 2)

    tile_ids = jnp.arange(n_expert_tiles, dtype=jnp.int32)
    te = jnp.sum((tile_ids[:, None] >= bounds[None, :]).astype(jnp.int32),
                 axis=1)
    te_last = jnp.sum((bounds <= n_act - 1).astype(jnp.int32))
    te = jnp.where(tile_ids < n_act, te, te_last)
    sh_te = E + jnp.arange(n_shared_tiles, dtype=jnp.int32) // (BT // TMG)
    info = jnp.concatenate([te, sh_te, n_act.reshape(1)])

    xg = flat.astype(jnp.bfloat16)[tok_pad]

    WgA = jnp.concatenate([Wg, Sg], axis=0)
    WuA = jnp.concatenate([Wu, Su], axis=0)
    WdA = jnp.concatenate([Wd, Sd], axis=0)

    grid_spec = pltpu.PrefetchScalarGridSpec(
        num_scalar_prefetch=1,
        grid=(n_tiles,),
        in_specs=[
            pl.BlockSpec((TMG, D), lambda i, info: (i, 0)),
            pl.BlockSpec((1, MID, D), lambda i, info: (info[i], 0, 0)),
            pl.BlockSpec((1, MID, D), lambda i, info: (info[i], 0, 0)),
            pl.BlockSpec((1, D, MID), lambda i, info: (info[i], 0, 0)),
        ],
        out_specs=pl.BlockSpec((TMG, D), lambda i, info: (i, 0)),
    )
    y = pl.pallas_call(
        functools.partial(_ffn_kernel, n_expert_tiles=n_expert_tiles,
                          n_tiles=n_tiles),
        grid_spec=grid_spec,
        out_shape=jax.ShapeDtypeStruct((padmax, D), jnp.float32),
        compiler_params=pltpu.CompilerParams(
            vmem_limit_bytes=64 * 1024 * 1024),
    )(info, xg, WgA, WuA, WdA)

    out = p2[:, 0:1] * y[pos2[:, 0]] + p2[:, 1:2] * y[pos2[:, 1]]
    for si in range(SH):
        base = shared_base + si * BT
        out = out + y[base:base + BT]

    return out.reshape(B, T, D), bal[0, 0]
